# Initial kernel scaffold; baseline (speedup 1.0000x reference)
import jax
import jax.numpy as jnp
from jax.experimental import pallas as pl


def kernel(enhanced_posts_embeddings, selected_reasoning_embeddings, llm_embeddings, gate_W, gate_b, U_W, U_b, V_W, V_b):
    raise NotImplementedError("write your pallas kernel here")


# speedup vs baseline: 1.6025x; 1.6025x over previous
import functools

import jax
import jax.numpy as jnp
from jax.experimental import pallas as pl
from jax.experimental.pallas import tpu as pltpu

B, D, H, R, K, NL = 4096, 384, 64, 8, 2, 64
AUX_COEF = 0.05
TILE = 512
GRID = B // TILE

_NEG = -3.0e38


def _dot_t(a, b):
    return jax.lax.dot_general(a, b, (((1,), (1,)), ((), ())),
                               preferred_element_type=jnp.float32)


def _moe_kernel(x1_ref, x2_ref, le_ref, gw_ref, gb_ref, uw_ref, ub_ref,
                vw_ref, vb_ref, rand_ref, sel_ref, logp_ref, aux_ref,
                accp_ref, accm_ref):
    i = pl.program_id(0)
    x1 = x1_ref[...]
    x2 = x2_ref[...]
    gw = gw_ref[...]

    logits = (_dot_t(x1, gw[:, :D]) + _dot_t(x2, gw[:, D:])
              + gb_ref[...])

    r_iota = jax.lax.broadcasted_iota(jnp.int32, logits.shape, 1)
    m1 = jnp.max(logits, axis=1, keepdims=True)
    i1 = jnp.min(jnp.where(logits == m1, r_iota, R), axis=1, keepdims=True)
    logits_m = jnp.where(r_iota == i1, _NEG, logits)
    m2 = jnp.max(logits_m, axis=1, keepdims=True)
    i2 = jnp.min(jnp.where(logits_m == m2, r_iota, R), axis=1, keepdims=True)

    e2 = jnp.exp(m2 - m1)
    w1 = 1.0 / (1.0 + e2)
    w2 = e2 / (1.0 + e2)

    p = jnp.exp(logits - m1)
    probs = p / jnp.sum(p, axis=1, keepdims=True)
    mask = ((r_iota == i1) | (r_iota == i2)).astype(jnp.float32)

    @pl.when(i == 0)
    def _():
        accp_ref[...] = jnp.zeros_like(accp_ref)
        accm_ref[...] = jnp.zeros_like(accm_ref)

    accp_ref[...] += jnp.sum(probs, axis=0, keepdims=True)
    accm_ref[...] += jnp.sum(mask, axis=0, keepdims=True)
    aux_ref[...] = (R * AUX_COEF / (B * B)) * jnp.sum(
        accp_ref[...] * accm_ref[...], axis=1, keepdims=True)

    le = le_ref[...]
    acc = jnp.zeros((x1.shape[0], NL), jnp.float32)
    for r in range(R):
        uw = uw_ref[r]
        ub = ub_ref[...][r]
        xh = _dot_t(x1, uw[:, :D]) + _dot_t(x2, uw[:, D:]) + ub[None, :]
        nx = jnp.sqrt(jnp.sum(xh * xh, axis=1, keepdims=True))
        xh = xh / jnp.maximum(nx, 1e-12)

        eh = _dot_t(le, vw_ref[r]) + vb_ref[...][r][None, :]
        ne = jnp.sqrt(jnp.sum(eh * eh, axis=1, keepdims=True))
        eh = eh / jnp.maximum(ne, 1e-12)

        s = _dot_t(xh, eh)
        s = s - jnp.max(s, axis=1, keepdims=True)
        s = jnp.exp(s)
        s = s / jnp.sum(s, axis=1, keepdims=True)

        w_r = jnp.where(i1 == r, w1, 0.0) + jnp.where(i2 == r, w2, 0.0)
        acc = acc + w_r * s

    tr = jax.lax.broadcasted_iota(jnp.int32, (NL, NL), 0)
    tc = jax.lax.broadcasted_iota(jnp.int32, (NL, NL), 1)
    tri = (tr <= tc).astype(jnp.float32)
    csum = jnp.dot(acc, tri, preferred_element_type=jnp.float32)
    rand = rand_ref[...]
    cnt = jnp.sum((csum <= rand).astype(jnp.int32), axis=1, keepdims=True)
    sel = jnp.where(cnt == NL, 0, cnt)
    sel_ref[...] = sel

    n_iota = jax.lax.broadcasted_iota(jnp.int32, acc.shape, 1)
    psel = jnp.sum(jnp.where(n_iota == sel, acc, 0.0), axis=1, keepdims=True)
    logp_ref[...] = jnp.log(psel)


@jax.jit
def kernel(enhanced_posts_embeddings, selected_reasoning_embeddings,
           llm_embeddings, gate_W, gate_b, U_W, U_b, V_W, V_b):
    gb2 = gate_b.reshape(1, R)
    rand = jax.random.uniform(jax.random.key(42), (B, 1))
    sel, logp, aux = pl.pallas_call(
        _moe_kernel,
        grid=(GRID,),
        in_specs=[
            pl.BlockSpec((TILE, D), lambda i: (i, 0)),
            pl.BlockSpec((TILE, D), lambda i: (i, 0)),
            pl.BlockSpec((NL, D), lambda i: (0, 0)),
            pl.BlockSpec((R, 2 * D), lambda i: (0, 0)),
            pl.BlockSpec((1, R), lambda i: (0, 0)),
            pl.BlockSpec((R, H, 2 * D), lambda i: (0, 0, 0)),
            pl.BlockSpec((R, H), lambda i: (0, 0)),
            pl.BlockSpec((R, H, D), lambda i: (0, 0, 0)),
            pl.BlockSpec((R, H), lambda i: (0, 0)),
            pl.BlockSpec((TILE, 1), lambda i: (i, 0)),
        ],
        out_specs=[
            pl.BlockSpec((TILE, 1), lambda i: (i, 0)),
            pl.BlockSpec((TILE, 1), lambda i: (i, 0)),
            pl.BlockSpec((1, 1), lambda i: (0, 0)),
        ],
        out_shape=[
            jax.ShapeDtypeStruct((B, 1), jnp.int32),
            jax.ShapeDtypeStruct((B, 1), jnp.float32),
            jax.ShapeDtypeStruct((1, 1), jnp.float32),
        ],
        scratch_shapes=[pltpu.VMEM((1, R), jnp.float32),
                        pltpu.VMEM((1, R), jnp.float32)],
    )(enhanced_posts_embeddings, selected_reasoning_embeddings,
      llm_embeddings, gate_W, gb2, U_W, U_b, V_W, V_b, rand)
    return sel[:, 0], logp, aux[0, 0]


# speedup vs baseline: 1.8362x; 1.1458x over previous
import jax
import jax.numpy as jnp
from jax.experimental import pallas as pl
from jax.experimental.pallas import tpu as pltpu

B, D, H, R, K, NL = 4096, 384, 64, 8, 2, 64
RH = R * H
AUX_COEF = 0.05
TILE = 512
GRID = B // TILE

_NEG = -3.0e38


def _moe_kernel(x1_ref, x2_ref, leT_ref, gwT_ref, gb_ref, uc_ref, ub_ref,
                vc_ref, vbc_ref, g_ref, tri_ref, f_ref, rand_ref,
                sel_ref, logp_ref, aux_ref, m_ref, accp_ref, accm_ref):
    i = pl.program_id(0)
    x1 = x1_ref[...]
    x2 = x2_ref[...]
    g = g_ref[...]

    @pl.when(i == 0)
    def _():
        eht = jax.lax.dot_general(vc_ref[...], leT_ref[...],
                                  (((0,), (0,)), ((), ())),
                                  preferred_element_type=jnp.float32)
        eht = eht + vbc_ref[...]
        en2 = jnp.dot(g, eht * eht, preferred_element_type=jnp.float32)
        ehn = eht / jnp.maximum(jnp.sqrt(en2), 1e-12)
        m_ref[...] = jnp.concatenate([ehn] * R, axis=1) * g
        accp_ref[...] = jnp.zeros_like(accp_ref)
        accm_ref[...] = jnp.zeros_like(accm_ref)

    gwt = gwT_ref[...]
    logits = (jnp.dot(x1, gwt[:D], preferred_element_type=jnp.float32)
              + jnp.dot(x2, gwt[D:], preferred_element_type=jnp.float32)
              + gb_ref[...])

    r_iota = jax.lax.broadcasted_iota(jnp.int32, logits.shape, 1)
    m1 = jnp.max(logits, axis=1, keepdims=True)
    i1 = jnp.min(jnp.where(logits == m1, r_iota, R), axis=1, keepdims=True)
    logits_m = jnp.where(r_iota == i1, _NEG, logits)
    m2 = jnp.max(logits_m, axis=1, keepdims=True)
    i2 = jnp.min(jnp.where(logits_m == m2, r_iota, R), axis=1, keepdims=True)

    e2 = jnp.exp(m2 - m1)
    w1 = 1.0 / (1.0 + e2)
    w2 = e2 / (1.0 + e2)

    p = jnp.exp(logits - m1)
    probs = p / jnp.sum(p, axis=1, keepdims=True)
    mask = ((r_iota == i1) | (r_iota == i2)).astype(jnp.float32)
    accp_ref[...] += jnp.sum(probs, axis=0, keepdims=True)
    accm_ref[...] += jnp.sum(mask, axis=0, keepdims=True)
    aux_ref[...] = (R * AUX_COEF / (B * B)) * jnp.sum(
        accp_ref[...] * accm_ref[...], axis=1, keepdims=True)

    xh = (jnp.dot(x1, uc_ref[:D], preferred_element_type=jnp.float32)
          + jnp.dot(x2, uc_ref[D:], preferred_element_type=jnp.float32)
          + ub_ref[...])
    n2 = jnp.dot(xh * xh, g, preferred_element_type=jnp.float32)
    xhn = xh / jnp.maximum(jnp.sqrt(n2), 1e-12)

    s = jnp.dot(xhn, m_ref[...], preferred_element_type=jnp.float32)
    es = jnp.exp(s)
    z = jnp.dot(es, g, preferred_element_type=jnp.float32)
    pr = es / z

    lane_r = jax.lax.broadcasted_iota(jnp.int32, pr.shape, 1) // NL
    w = jnp.where(lane_r == i1, w1, 0.0) + jnp.where(lane_r == i2, w2, 0.0)

    llm_probs = jnp.dot(pr * w, f_ref[...], preferred_element_type=jnp.float32)

    csum = jnp.dot(llm_probs, tri_ref[...], preferred_element_type=jnp.float32)
    rand = rand_ref[...]
    cnt = jnp.sum((csum <= rand).astype(jnp.int32), axis=1, keepdims=True)
    sel = jnp.where(cnt == NL, 0, cnt)
    sel_ref[...] = sel

    n_iota = jax.lax.broadcasted_iota(jnp.int32, llm_probs.shape, 1)
    psel = jnp.sum(jnp.where(n_iota == sel, llm_probs, 0.0),
                   axis=1, keepdims=True)
    logp_ref[...] = jnp.log(psel)


@jax.jit
def kernel(enhanced_posts_embeddings, selected_reasoning_embeddings,
           llm_embeddings, gate_W, gate_b, U_W, U_b, V_W, V_b):
    uc = U_W.transpose(2, 0, 1).reshape(2 * D, RH)
    ub = U_b.reshape(1, RH)
    vc = V_W.transpose(2, 0, 1).reshape(D, RH)
    vbc = V_b.reshape(RH, 1)
    gwT = gate_W.T
    gb2 = gate_b.reshape(1, R)
    leT = llm_embeddings.T
    rand = jax.random.uniform(jax.random.key(42), (B, 1))

    ri = jnp.arange(RH)
    g_blk = (ri[:, None] // H == ri[None, :] // H).astype(jnp.float32)
    nn = jnp.arange(NL)
    tri = (nn[:, None] <= nn[None, :]).astype(jnp.float32)
    f_fold = (ri[:, None] % NL == nn[None, :]).astype(jnp.float32)

    const_spec2 = lambda shape: pl.BlockSpec(shape, lambda i: (0, 0))
    sel, logp, aux = pl.pallas_call(
        _moe_kernel,
        grid=(GRID,),
        in_specs=[
            pl.BlockSpec((TILE, D), lambda i: (i, 0)),
            pl.BlockSpec((TILE, D), lambda i: (i, 0)),
            const_spec2((D, NL)),
            const_spec2((2 * D, R)),
            const_spec2((1, R)),
            const_spec2((2 * D, RH)),
            const_spec2((1, RH)),
            const_spec2((D, RH)),
            const_spec2((RH, 1)),
            const_spec2((RH, RH)),
            const_spec2((NL, NL)),
            const_spec2((RH, NL)),
            pl.BlockSpec((TILE, 1), lambda i: (i, 0)),
        ],
        out_specs=[
            pl.BlockSpec((TILE, 1), lambda i: (i, 0)),
            pl.BlockSpec((TILE, 1), lambda i: (i, 0)),
            pl.BlockSpec((1, 1), lambda i: (0, 0)),
        ],
        out_shape=[
            jax.ShapeDtypeStruct((B, 1), jnp.int32),
            jax.ShapeDtypeStruct((B, 1), jnp.float32),
            jax.ShapeDtypeStruct((1, 1), jnp.float32),
        ],
        scratch_shapes=[pltpu.VMEM((RH, RH), jnp.float32),
                        pltpu.VMEM((1, R), jnp.float32),
                        pltpu.VMEM((1, R), jnp.float32)],
    )(enhanced_posts_embeddings, selected_reasoning_embeddings,
      leT, gwT, gb2, uc, ub, vc, vbc, g_blk, tri, f_fold, rand)
    return sel[:, 0], logp, aux[0, 0]


# speedup vs baseline: 1.8898x; 1.0292x over previous
import jax
import jax.numpy as jnp
from jax.experimental import pallas as pl
from jax.experimental.pallas import tpu as pltpu

B, D, H, R, K, NL = 4096, 384, 64, 8, 2, 64
RH = R * H
AUX_COEF = 0.05
TILE = 1024
GRID = B // TILE

_NEG = -3.0e38


def _moe_kernel(x1_ref, x2_ref, leT_ref, gwT_ref, gb_ref, uc_ref, ub_ref,
                vc_ref, vbc_ref, g_ref, tri_ref, f_ref, rand_ref,
                sel_ref, logp_ref, aux_ref, m_ref, accp_ref, accm_ref):
    i = pl.program_id(0)
    x1 = x1_ref[...]
    x2 = x2_ref[...]
    g = g_ref[...]

    @pl.when(i == 0)
    def _():
        eht = jax.lax.dot_general(vc_ref[...], leT_ref[...],
                                  (((0,), (0,)), ((), ())),
                                  preferred_element_type=jnp.float32)
        eht = eht + vbc_ref[...]
        en2 = jnp.dot(g, eht * eht, preferred_element_type=jnp.float32)
        ehn = eht / jnp.maximum(jnp.sqrt(en2), 1e-12)
        m_ref[...] = jnp.concatenate([ehn] * R, axis=1) * g
        accp_ref[...] = jnp.zeros_like(accp_ref)
        accm_ref[...] = jnp.zeros_like(accm_ref)

    gwt = gwT_ref[...]
    logits = (jnp.dot(x1, gwt[:D], preferred_element_type=jnp.float32)
              + jnp.dot(x2, gwt[D:], preferred_element_type=jnp.float32)
              + gb_ref[...])

    r_iota = jax.lax.broadcasted_iota(jnp.int32, logits.shape, 1)
    m1 = jnp.max(logits, axis=1, keepdims=True)
    i1 = jnp.min(jnp.where(logits == m1, r_iota, R), axis=1, keepdims=True)
    logits_m = jnp.where(r_iota == i1, _NEG, logits)
    m2 = jnp.max(logits_m, axis=1, keepdims=True)
    i2 = jnp.min(jnp.where(logits_m == m2, r_iota, R), axis=1, keepdims=True)

    e2 = jnp.exp(m2 - m1)
    w1 = 1.0 / (1.0 + e2)
    w2 = e2 / (1.0 + e2)

    p = jnp.exp(logits - m1)
    probs = p / jnp.sum(p, axis=1, keepdims=True)
    mask = ((r_iota == i1) | (r_iota == i2)).astype(jnp.float32)
    accp_ref[...] += jnp.sum(probs, axis=0, keepdims=True)
    accm_ref[...] += jnp.sum(mask, axis=0, keepdims=True)
    aux_ref[...] = (R * AUX_COEF / (B * B)) * jnp.sum(
        accp_ref[...] * accm_ref[...], axis=1, keepdims=True)

    xh = (jnp.dot(x1, uc_ref[:D], preferred_element_type=jnp.float32)
          + jnp.dot(x2, uc_ref[D:], preferred_element_type=jnp.float32)
          + ub_ref[...])
    n2 = jnp.dot(xh * xh, g, preferred_element_type=jnp.float32)
    xhn = xh / jnp.maximum(jnp.sqrt(n2), 1e-12)

    s = jnp.dot(xhn, m_ref[...], preferred_element_type=jnp.float32)
    es = jnp.exp(s)
    z = jnp.dot(es, g, preferred_element_type=jnp.float32)
    pr = es / z

    lane_r = jax.lax.broadcasted_iota(jnp.int32, pr.shape, 1) // NL
    w = jnp.where(lane_r == i1, w1, 0.0) + jnp.where(lane_r == i2, w2, 0.0)

    llm_probs = jnp.dot(pr * w, f_ref[...], preferred_element_type=jnp.float32)

    csum = jnp.dot(llm_probs, tri_ref[...], preferred_element_type=jnp.float32)
    rand = rand_ref[...]
    cnt = jnp.sum((csum <= rand).astype(jnp.int32), axis=1, keepdims=True)
    sel = jnp.where(cnt == NL, 0, cnt)
    sel_ref[...] = sel

    n_iota = jax.lax.broadcasted_iota(jnp.int32, llm_probs.shape, 1)
    psel = jnp.sum(jnp.where(n_iota == sel, llm_probs, 0.0),
                   axis=1, keepdims=True)
    logp_ref[...] = jnp.log(psel)


@jax.jit
def kernel(enhanced_posts_embeddings, selected_reasoning_embeddings,
           llm_embeddings, gate_W, gate_b, U_W, U_b, V_W, V_b):
    uc = U_W.transpose(2, 0, 1).reshape(2 * D, RH)
    ub = U_b.reshape(1, RH)
    vc = V_W.transpose(2, 0, 1).reshape(D, RH)
    vbc = V_b.reshape(RH, 1)
    gwT = gate_W.T
    gb2 = gate_b.reshape(1, R)
    leT = llm_embeddings.T
    rand = jax.random.uniform(jax.random.key(42), (B, 1))

    ri = jnp.arange(RH)
    g_blk = (ri[:, None] // H == ri[None, :] // H).astype(jnp.float32)
    nn = jnp.arange(NL)
    tri = (nn[:, None] <= nn[None, :]).astype(jnp.float32)
    f_fold = (ri[:, None] % NL == nn[None, :]).astype(jnp.float32)

    const_spec2 = lambda shape: pl.BlockSpec(shape, lambda i: (0, 0))
    sel, logp, aux = pl.pallas_call(
        _moe_kernel,
        grid=(GRID,),
        in_specs=[
            pl.BlockSpec((TILE, D), lambda i: (i, 0)),
            pl.BlockSpec((TILE, D), lambda i: (i, 0)),
            const_spec2((D, NL)),
            const_spec2((2 * D, R)),
            const_spec2((1, R)),
            const_spec2((2 * D, RH)),
            const_spec2((1, RH)),
            const_spec2((D, RH)),
            const_spec2((RH, 1)),
            const_spec2((RH, RH)),
            const_spec2((NL, NL)),
            const_spec2((RH, NL)),
            pl.BlockSpec((TILE, 1), lambda i: (i, 0)),
        ],
        out_specs=[
            pl.BlockSpec((TILE, 1), lambda i: (i, 0)),
            pl.BlockSpec((TILE, 1), lambda i: (i, 0)),
            pl.BlockSpec((1, 1), lambda i: (0, 0)),
        ],
        out_shape=[
            jax.ShapeDtypeStruct((B, 1), jnp.int32),
            jax.ShapeDtypeStruct((B, 1), jnp.float32),
            jax.ShapeDtypeStruct((1, 1), jnp.float32),
        ],
        scratch_shapes=[pltpu.VMEM((RH, RH), jnp.float32),
                        pltpu.VMEM((1, R), jnp.float32),
                        pltpu.VMEM((1, R), jnp.float32)],
    )(enhanced_posts_embeddings, selected_reasoning_embeddings,
      leT, gwT, gb2, uc, ub, vc, vbc, g_blk, tri, f_fold, rand)
    return sel[:, 0], logp, aux[0, 0]
